# baseline (speedup 1.0000x reference)
import jax
import jax.numpy as jnp
from jax import lax
from jax.experimental import pallas as pl
from jax.experimental.pallas import tpu as pltpu
from jax.experimental.pallas import tpu_sc as plsc

_K = 64
_ROWS = 128
_COLS = 32768
_BLOCK_ROWS = 48
_INT_MIN = -2147483648
_BIG = 1 << 30
_NW = 32
_SC_ROWS = 32
_NVR = _COLS // 16
_UNROLL = 16

_NOISE_CACHE = []


def _gumbel_noise(slot=0):
    if not _NOISE_CACHE:
        try:
            with jax.ensure_compile_time_eval():
                noise = jax.random.gumbel(
                    jax.random.key(42), (_ROWS, _COLS), jnp.float32)
                _NOISE_CACHE.append(noise)
                _NOISE_CACHE.append(noise + jnp.zeros((), jnp.float32))
        except Exception:
            return jax.random.gumbel(
                jax.random.key(42), (_ROWS, _COLS), jnp.float32)
    return _NOISE_CACHE[slot]


def _row_count(mask):
    return jnp.sum(mask.astype(jnp.int32), axis=1, keepdims=True)


def _tc_body(tau_ref, scores_ref, noise_ref, out_ref):
    nrows = scores_ref.shape[0]
    tau = tau_ref[0]
    g = scores_ref[...] / tau + noise_ref[...]
    u = lax.bitcast_convert_type(g, jnp.int32)
    s = u ^ ((u >> 31) & jnp.int32(0x7FFFFFFF))

    colmax = jnp.max(s.reshape(nrows, _COLS // 128, 128), axis=1)
    g1 = jnp.max(colmax, axis=1, keepdims=True)

    def ccount(t):
        return jnp.sum((colmax >= t).astype(jnp.int32), axis=1, keepdims=True)

    base0 = jnp.where(ccount(jnp.zeros((nrows, 1), jnp.int32)) >= _K,
                      0, _INT_MIN).astype(jnp.int32)

    def c_step(i, b):
        cand = b + (jnp.int32(1) << (jnp.int32(30) - i))
        ok = jnp.logical_and(ccount(cand) >= _K, cand >= b)
        return jnp.where(ok, cand, b)

    t0 = lax.fori_loop(0, 31, c_step, base0)

    rangef = g1.astype(jnp.float32) - t0.astype(jnp.float32)
    rmax = jnp.max(rangef)
    e = (lax.bitcast_convert_type(jnp.maximum(rmax, 1.0), jnp.int32)
         >> 23) - 126
    nbits = jnp.clip(e + 2, 1, 31)

    def vstep(i, b):
        cand = b + (jnp.int32(1) << (nbits - 1 - i))
        ok = jnp.logical_and(_row_count(s >= cand) >= _K, cand >= b)
        return jnp.where(ok, cand, b)

    thr = lax.fori_loop(0, nbits, vstep, t0)

    gt = s > thr
    eq = s == thr
    need = _K - _row_count(gt)
    c_eq = _row_count(eq)
    idx = lax.broadcasted_iota(jnp.int32, g.shape, 1)

    def tie_descent(_):
        def istep(i, b):
            cand = b + (jnp.int32(1) << (jnp.int32(14) - i))
            c = _row_count(eq & (idx <= cand))
            return jnp.where(c <= need, cand, b)
        return lax.fori_loop(0, 15, istep, jnp.full_like(need, -1))

    easy = jnp.all(c_eq == need)
    tie_idx = lax.cond(
        easy, lambda _: jnp.full_like(need, _COLS - 1), tie_descent, 0)

    mask = gt | (eq & (idx <= tie_idx))
    out_ref[...] = mask.astype(jnp.float32)


def _tc_call(scores, tau, noise):
    off = _SC_ROWS // _BLOCK_ROWS
    grid = ((_ROWS - _SC_ROWS) // _BLOCK_ROWS,)
    return pl.pallas_call(
        _tc_body,
        grid=grid,
        in_specs=[
            pl.BlockSpec(memory_space=pltpu.SMEM),
            pl.BlockSpec((_BLOCK_ROWS, _COLS), lambda i: (i + off, 0)),
            pl.BlockSpec((_BLOCK_ROWS, _COLS), lambda i: (i + off, 0)),
        ],
        out_specs=pl.BlockSpec((_BLOCK_ROWS, _COLS), lambda i: (i + off, 0)),
        out_shape=jax.ShapeDtypeStruct((_ROWS, _COLS), jnp.float32),
        compiler_params=pltpu.CompilerParams(
            dimension_semantics=("arbitrary",),
        ),
    )(tau, scores, noise)


def _take16(x, idx):
    dn = lax.GatherDimensionNumbers(
        offset_dims=(), collapsed_slice_dims=(0,), start_index_map=(0,))
    return lax.gather(x, idx[:, None], dn, (1,),
                      mode=lax.GatherScatterMode.PROMISE_IN_BOUNDS)


def _sc_body(tau_hbm, scores_hbm, noise_hbm, out_hbm,
             key_buf, aux_buf, sent_buf, tau_buf):
    rows_per_w = _SC_ROWS // _NW
    wid = lax.axis_index("s") * 2 + lax.axis_index("c")
    iota = lax.iota(jnp.int32, 16)
    zero_i = jnp.zeros((16,), jnp.int32)
    one_i = jnp.full((16,), 1, jnp.int32)
    k_spl = jnp.full((16,), _K, jnp.int32)
    zero_f = jnp.zeros((16,), jnp.float32)
    one_f = jnp.full((16,), 1.0, jnp.float32)

    pltpu.sync_copy(tau_hbm, tau_buf)
    tauv = tau_buf[...]

    def allreduce(v):
        for d in (1, 2, 4, 8):
            v = v + _take16(v, iota ^ d)
        return v

    def key_at(v):
        return lax.bitcast_convert_type(key_buf[pl.ds(v * 16, 16)], jnp.int32)

    def count_ge(t_spl):
        def cbody(j, accs):
            accs = list(accs)
            for k in range(_UNROLL):
                kv = key_at(j * _UNROLL + k)
                accs[k] = accs[k] + jnp.where(kv >= t_spl, one_i, zero_i)
            return tuple(accs)
        accs = lax.fori_loop(0, _NVR // _UNROLL, cbody, (zero_i,) * _UNROLL)
        acc = accs[0]
        for k in range(1, _UNROLL):
            acc = acc + accs[k]
        return allreduce(acc)

    for i in range(rows_per_w):
        r = wid * rows_per_w + i
        pltpu.sync_copy(scores_hbm.at[r], key_buf)
        pltpu.sync_copy(noise_hbm.at[r], aux_buf)

        def p1(j, carry):
            for k in range(_UNROLL):
                sl = pl.ds((j * _UNROLL + k) * 16, 16)
                gv = key_buf[sl] / tauv + aux_buf[sl]
                u = lax.bitcast_convert_type(gv, jnp.int32)
                key = u ^ (lax.shift_right_arithmetic(u, 31) & 0x7FFFFFFF)
                key_buf[sl] = lax.bitcast_convert_type(key, jnp.float32)
            return carry
        lax.fori_loop(0, _NVR // _UNROLL, p1, 0)

        c0 = count_ge(zero_i)
        base = jnp.where(c0 >= k_spl, zero_i, jnp.full((16,), _INT_MIN,
                                                       jnp.int32))

        def vstep(it, b):
            cand = b + (jnp.int32(1) << (jnp.int32(30) - it))
            c = count_ge(cand)
            ok = jnp.logical_and(c >= k_spl, cand >= b)
            return jnp.where(ok, cand, b)
        thr = lax.fori_loop(0, 31, vstep, base)

        def p2(j, accs):
            accs = list(accs)
            for k in range(_UNROLL):
                v = j * _UNROLL + k
                kv = key_at(v)
                accs[k] = accs[k] + jnp.where(kv > thr, one_i, zero_i)
                idxv = iota + v * 16
                sent_buf[pl.ds(v * 16, 16)] = jnp.where(
                    kv == thr, idxv, jnp.full((16,), _BIG, jnp.int32))
            return tuple(accs)
        accs = lax.fori_loop(0, _NVR // _UNROLL, p2, (zero_i,) * _UNROLL)
        acc = accs[0]
        for k in range(1, _UNROLL):
            acc = acc + accs[k]
        need = k_spl - allreduce(acc)

        def istep(it, b):
            candI = b + (jnp.int32(1) << (jnp.int32(14) - it))

            def ibody(j, accs):
                accs = list(accs)
                for k in range(_UNROLL):
                    sv = sent_buf[pl.ds((j * _UNROLL + k) * 16, 16)]
                    accs[k] = accs[k] + jnp.where(sv <= candI, one_i, zero_i)
                return tuple(accs)
            accs = lax.fori_loop(0, _NVR // _UNROLL, ibody,
                                 (zero_i,) * _UNROLL)
            acc = accs[0]
            for k in range(1, _UNROLL):
                acc = acc + accs[k]
            c = allreduce(acc)
            return jnp.where(c <= need, candI, b)
        tieI = lax.fori_loop(0, 15, istep, jnp.full((16,), -1, jnp.int32))

        def p3(j, carry):
            for k in range(_UNROLL):
                v = j * _UNROLL + k
                kv = key_at(v)
                sv = sent_buf[pl.ds(v * 16, 16)]
                sel = jnp.logical_or(kv > thr, sv <= tieI)
                aux_buf[pl.ds(v * 16, 16)] = jnp.where(sel, one_f, zero_f)
            return carry
        lax.fori_loop(0, _NVR // _UNROLL, p3, 0)
        pltpu.sync_copy(aux_buf, out_hbm.at[r])


def _sc_call(scores, tau16, noise):
    mesh = plsc.VectorSubcoreMesh(core_axis_name="c", subcore_axis_name="s")
    f = pl.kernel(
        _sc_body,
        out_type=jax.ShapeDtypeStruct((_SC_ROWS, _COLS), jnp.float32),
        mesh=mesh,
        scratch_types=[
            pltpu.VMEM((_COLS,), jnp.float32),
            pltpu.VMEM((_COLS,), jnp.float32),
            pltpu.VMEM((_COLS,), jnp.int32),
            pltpu.VMEM((16,), jnp.float32),
        ],
    )
    return f(tau16, scores, noise)


def kernel(scores, tau):
    if _SC_ROWS == _ROWS:
        tau16 = jnp.broadcast_to(tau.astype(jnp.float32), (16,))
        return _sc_call(scores, tau16, _gumbel_noise(1))
    if _SC_ROWS == 0:
        return _tc_call(scores, tau, _gumbel_noise(0))
    tau16 = jnp.broadcast_to(tau.astype(jnp.float32), (16,))
    sc_out = _sc_call(scores, tau16, _gumbel_noise(1))
    tc_out = _tc_call(scores, tau, _gumbel_noise(0))
    return lax.dynamic_update_slice(tc_out, sc_out, (0, 0))


# speedup vs baseline: 1.0070x; 1.0070x over previous
import jax
import jax.numpy as jnp
from jax import lax
from jax.experimental import pallas as pl
from jax.experimental.pallas import tpu as pltpu
from jax.experimental.pallas import tpu_sc as plsc

_K = 64
_ROWS = 128
_COLS = 32768
_BLOCK_ROWS = 32
_INT_MIN = -2147483648
_BIG = 1 << 30
_NW = 32
_SC_ROWS = 32
_NVR = _COLS // 16
_UNROLL = 16

_NOISE_CACHE = []


def _gumbel_noise(slot=0):
    if not _NOISE_CACHE:
        try:
            with jax.ensure_compile_time_eval():
                noise = jax.random.gumbel(
                    jax.random.key(42), (_ROWS, _COLS), jnp.float32)
                _NOISE_CACHE.append(noise)
                _NOISE_CACHE.append(noise + jnp.zeros((), jnp.float32))
        except Exception:
            return jax.random.gumbel(
                jax.random.key(42), (_ROWS, _COLS), jnp.float32)
    return _NOISE_CACHE[slot]


def _row_count(mask):
    return jnp.sum(mask.astype(jnp.int32), axis=1, keepdims=True)


def _tc_body(tau_ref, scores_ref, noise_ref, out_ref):
    nrows = scores_ref.shape[0]
    tau = tau_ref[0]
    g = scores_ref[...] / tau + noise_ref[...]
    u = lax.bitcast_convert_type(g, jnp.int32)
    s = u ^ ((u >> 31) & jnp.int32(0x7FFFFFFF))

    colmax = jnp.max(s.reshape(nrows, _COLS // 128, 128), axis=1)
    g1 = jnp.max(colmax, axis=1, keepdims=True)

    def ccount(t):
        return jnp.sum((colmax >= t).astype(jnp.int32), axis=1, keepdims=True)

    base0 = jnp.where(ccount(jnp.zeros((nrows, 1), jnp.int32)) >= _K,
                      0, _INT_MIN).astype(jnp.int32)

    def c_step(i, b):
        cand = b + (jnp.int32(1) << (jnp.int32(30) - i))
        ok = jnp.logical_and(ccount(cand) >= _K, cand >= b)
        return jnp.where(ok, cand, b)

    t0 = lax.fori_loop(0, 31, c_step, base0)

    rangef = g1.astype(jnp.float32) - t0.astype(jnp.float32)
    rmax = jnp.max(rangef)
    e = (lax.bitcast_convert_type(jnp.maximum(rmax, 1.0), jnp.int32)
         >> 23) - 126
    nbits = jnp.clip(e + 2, 1, 31)

    def vstep(i, b):
        cand = b + (jnp.int32(1) << (nbits - 1 - i))
        ok = jnp.logical_and(_row_count(s >= cand) >= _K, cand >= b)
        return jnp.where(ok, cand, b)

    thr = lax.fori_loop(0, nbits, vstep, t0)

    gt = s > thr
    eq = s == thr
    need = _K - _row_count(gt)
    c_eq = _row_count(eq)
    idx = lax.broadcasted_iota(jnp.int32, g.shape, 1)

    def tie_descent(_):
        def istep(i, b):
            cand = b + (jnp.int32(1) << (jnp.int32(14) - i))
            c = _row_count(eq & (idx <= cand))
            return jnp.where(c <= need, cand, b)
        return lax.fori_loop(0, 15, istep, jnp.full_like(need, -1))

    easy = jnp.all(c_eq == need)
    tie_idx = lax.cond(
        easy, lambda _: jnp.full_like(need, _COLS - 1), tie_descent, 0)

    mask = gt | (eq & (idx <= tie_idx))
    out_ref[...] = mask.astype(jnp.float32)


def _tc_call(scores, tau, noise):
    off = _SC_ROWS // _BLOCK_ROWS
    grid = ((_ROWS - _SC_ROWS) // _BLOCK_ROWS,)
    return pl.pallas_call(
        _tc_body,
        grid=grid,
        in_specs=[
            pl.BlockSpec(memory_space=pltpu.SMEM),
            pl.BlockSpec((_BLOCK_ROWS, _COLS), lambda i: (i + off, 0)),
            pl.BlockSpec((_BLOCK_ROWS, _COLS), lambda i: (i + off, 0)),
        ],
        out_specs=pl.BlockSpec((_BLOCK_ROWS, _COLS), lambda i: (i + off, 0)),
        out_shape=jax.ShapeDtypeStruct((_ROWS, _COLS), jnp.float32),
        compiler_params=pltpu.CompilerParams(
            dimension_semantics=("arbitrary",),
        ),
    )(tau, scores, noise)


def _take16(x, idx):
    dn = lax.GatherDimensionNumbers(
        offset_dims=(), collapsed_slice_dims=(0,), start_index_map=(0,))
    return lax.gather(x, idx[:, None], dn, (1,),
                      mode=lax.GatherScatterMode.PROMISE_IN_BOUNDS)


def _sc_body(tau_hbm, scores_hbm, noise_hbm, out_hbm,
             key_buf, aux_buf, sent_buf, tau_buf):
    rows_per_w = _SC_ROWS // _NW
    wid = lax.axis_index("s") * 2 + lax.axis_index("c")
    iota = lax.iota(jnp.int32, 16)
    zero_i = jnp.zeros((16,), jnp.int32)
    one_i = jnp.full((16,), 1, jnp.int32)
    k_spl = jnp.full((16,), _K, jnp.int32)
    zero_f = jnp.zeros((16,), jnp.float32)
    one_f = jnp.full((16,), 1.0, jnp.float32)

    pltpu.sync_copy(tau_hbm, tau_buf)
    tauv = tau_buf[...]

    def allreduce(v):
        for d in (1, 2, 4, 8):
            v = v + _take16(v, iota ^ d)
        return v

    def key_at(v):
        return lax.bitcast_convert_type(key_buf[pl.ds(v * 16, 16)], jnp.int32)

    def count_ge(t_spl):
        def cbody(j, accs):
            accs = list(accs)
            for k in range(_UNROLL):
                kv = key_at(j * _UNROLL + k)
                accs[k] = accs[k] + jnp.where(kv >= t_spl, one_i, zero_i)
            return tuple(accs)
        accs = lax.fori_loop(0, _NVR // _UNROLL, cbody, (zero_i,) * _UNROLL)
        acc = accs[0]
        for k in range(1, _UNROLL):
            acc = acc + accs[k]
        return allreduce(acc)

    for i in range(rows_per_w):
        r = wid * rows_per_w + i
        pltpu.sync_copy(scores_hbm.at[r], key_buf)
        pltpu.sync_copy(noise_hbm.at[r], aux_buf)

        def p1(j, carry):
            for k in range(_UNROLL):
                sl = pl.ds((j * _UNROLL + k) * 16, 16)
                gv = key_buf[sl] / tauv + aux_buf[sl]
                u = lax.bitcast_convert_type(gv, jnp.int32)
                key = u ^ (lax.shift_right_arithmetic(u, 31) & 0x7FFFFFFF)
                key_buf[sl] = lax.bitcast_convert_type(key, jnp.float32)
            return carry
        lax.fori_loop(0, _NVR // _UNROLL, p1, 0)

        c0 = count_ge(zero_i)
        base = jnp.where(c0 >= k_spl, zero_i, jnp.full((16,), _INT_MIN,
                                                       jnp.int32))

        def vstep(it, b):
            cand = b + (jnp.int32(1) << (jnp.int32(30) - it))
            c = count_ge(cand)
            ok = jnp.logical_and(c >= k_spl, cand >= b)
            return jnp.where(ok, cand, b)
        thr = lax.fori_loop(0, 31, vstep, base)

        def p2(j, accs):
            accs = list(accs)
            for k in range(_UNROLL):
                v = j * _UNROLL + k
                kv = key_at(v)
                accs[k] = accs[k] + jnp.where(kv > thr, one_i, zero_i)
                idxv = iota + v * 16
                sent_buf[pl.ds(v * 16, 16)] = jnp.where(
                    kv == thr, idxv, jnp.full((16,), _BIG, jnp.int32))
            return tuple(accs)
        accs = lax.fori_loop(0, _NVR // _UNROLL, p2, (zero_i,) * _UNROLL)
        acc = accs[0]
        for k in range(1, _UNROLL):
            acc = acc + accs[k]
        need = k_spl - allreduce(acc)

        def istep(it, b):
            candI = b + (jnp.int32(1) << (jnp.int32(14) - it))

            def ibody(j, accs):
                accs = list(accs)
                for k in range(_UNROLL):
                    sv = sent_buf[pl.ds((j * _UNROLL + k) * 16, 16)]
                    accs[k] = accs[k] + jnp.where(sv <= candI, one_i, zero_i)
                return tuple(accs)
            accs = lax.fori_loop(0, _NVR // _UNROLL, ibody,
                                 (zero_i,) * _UNROLL)
            acc = accs[0]
            for k in range(1, _UNROLL):
                acc = acc + accs[k]
            c = allreduce(acc)
            return jnp.where(c <= need, candI, b)
        tieI = lax.fori_loop(0, 15, istep, jnp.full((16,), -1, jnp.int32))

        def p3(j, carry):
            for k in range(_UNROLL):
                v = j * _UNROLL + k
                kv = key_at(v)
                sv = sent_buf[pl.ds(v * 16, 16)]
                sel = jnp.logical_or(kv > thr, sv <= tieI)
                aux_buf[pl.ds(v * 16, 16)] = jnp.where(sel, one_f, zero_f)
            return carry
        lax.fori_loop(0, _NVR // _UNROLL, p3, 0)
        pltpu.sync_copy(aux_buf, out_hbm.at[r])


def _sc_call(scores, tau16, noise):
    mesh = plsc.VectorSubcoreMesh(core_axis_name="c", subcore_axis_name="s")
    f = pl.kernel(
        _sc_body,
        out_type=jax.ShapeDtypeStruct((_SC_ROWS, _COLS), jnp.float32),
        mesh=mesh,
        scratch_types=[
            pltpu.VMEM((_COLS,), jnp.float32),
            pltpu.VMEM((_COLS,), jnp.float32),
            pltpu.VMEM((_COLS,), jnp.int32),
            pltpu.VMEM((16,), jnp.float32),
        ],
    )
    return f(tau16, scores, noise)


def kernel(scores, tau):
    if _SC_ROWS == _ROWS:
        tau16 = jnp.broadcast_to(tau.astype(jnp.float32), (16,))
        return _sc_call(scores, tau16, _gumbel_noise(1))
    if _SC_ROWS == 0:
        return _tc_call(scores, tau, _gumbel_noise(0))
    tau16 = jnp.broadcast_to(tau.astype(jnp.float32), (16,))
    sc_out = _sc_call(scores, tau16, _gumbel_noise(1))
    tc_out = _tc_call(scores, tau, _gumbel_noise(0))
    return lax.dynamic_update_slice(tc_out, sc_out, (0, 0))


# speedup vs baseline: 1.0394x; 1.0321x over previous
import jax
import jax.numpy as jnp
from jax import lax
from jax.experimental import pallas as pl
from jax.experimental.pallas import tpu as pltpu
from jax.experimental.pallas import tpu_sc as plsc

_K = 64
_ROWS = 128
_COLS = 32768
_BLOCK_ROWS = 32
_INT_MIN = -2147483648
_BIG = 1 << 30
_NW = 32
_SC_ROWS = 32
_NVR = _COLS // 16
_UNROLL = 16

_NOISE_CACHE = []


def _gumbel_noise(slot=0):
    if not _NOISE_CACHE:
        try:
            with jax.ensure_compile_time_eval():
                noise = jax.random.gumbel(
                    jax.random.key(42), (_ROWS, _COLS), jnp.float32)
                _NOISE_CACHE.append(noise)
                _NOISE_CACHE.append(noise[:_SC_ROWS] + jnp.zeros((), jnp.float32))
        except Exception:
            return jax.random.gumbel(
                jax.random.key(42), (_ROWS, _COLS), jnp.float32)
    return _NOISE_CACHE[slot]


def _row_count(mask):
    return jnp.sum(mask.astype(jnp.int32), axis=1, keepdims=True)


def _tc_body(tau_ref, scores_ref, noise_ref, out_ref):
    nrows = scores_ref.shape[0]
    tau = tau_ref[0]
    g = scores_ref[...] / tau + noise_ref[...]
    u = lax.bitcast_convert_type(g, jnp.int32)
    s = u ^ ((u >> 31) & jnp.int32(0x7FFFFFFF))

    colmax = jnp.max(s.reshape(nrows, _COLS // 128, 128), axis=1)
    g1 = jnp.max(colmax, axis=1, keepdims=True)

    def ccount(t):
        return jnp.sum((colmax >= t).astype(jnp.int32), axis=1, keepdims=True)

    base0 = jnp.where(ccount(jnp.zeros((nrows, 1), jnp.int32)) >= _K,
                      0, _INT_MIN).astype(jnp.int32)

    def c_step(i, b):
        cand = b + (jnp.int32(1) << (jnp.int32(30) - i))
        ok = jnp.logical_and(ccount(cand) >= _K, cand >= b)
        return jnp.where(ok, cand, b)

    t0 = lax.fori_loop(0, 31, c_step, base0)

    rangef = g1.astype(jnp.float32) - t0.astype(jnp.float32)
    rmax = jnp.max(rangef)
    e = (lax.bitcast_convert_type(jnp.maximum(rmax, 1.0), jnp.int32)
         >> 23) - 126
    nbits = jnp.clip(e + 2, 1, 31)

    def vstep(i, b):
        cand = b + (jnp.int32(1) << (nbits - 1 - i))
        ok = jnp.logical_and(_row_count(s >= cand) >= _K, cand >= b)
        return jnp.where(ok, cand, b)

    thr = lax.fori_loop(0, nbits, vstep, t0)

    gt = s > thr
    eq = s == thr
    need = _K - _row_count(gt)
    c_eq = _row_count(eq)
    idx = lax.broadcasted_iota(jnp.int32, g.shape, 1)

    def tie_descent(_):
        def istep(i, b):
            cand = b + (jnp.int32(1) << (jnp.int32(14) - i))
            c = _row_count(eq & (idx <= cand))
            return jnp.where(c <= need, cand, b)
        return lax.fori_loop(0, 15, istep, jnp.full_like(need, -1))

    easy = jnp.all(c_eq == need)
    tie_idx = lax.cond(
        easy, lambda _: jnp.full_like(need, _COLS - 1), tie_descent, 0)

    mask = gt | (eq & (idx <= tie_idx))
    out_ref[...] = mask.astype(jnp.float32)


def _tc_call(scores, tau, noise):
    off = _SC_ROWS // _BLOCK_ROWS
    grid = ((_ROWS - _SC_ROWS) // _BLOCK_ROWS,)
    return pl.pallas_call(
        _tc_body,
        grid=grid,
        in_specs=[
            pl.BlockSpec(memory_space=pltpu.SMEM),
            pl.BlockSpec((_BLOCK_ROWS, _COLS), lambda i: (i + off, 0)),
            pl.BlockSpec((_BLOCK_ROWS, _COLS), lambda i: (i + off, 0)),
        ],
        out_specs=pl.BlockSpec((_BLOCK_ROWS, _COLS), lambda i: (i + off, 0)),
        out_shape=jax.ShapeDtypeStruct((_ROWS, _COLS), jnp.float32),
        compiler_params=pltpu.CompilerParams(
            dimension_semantics=("arbitrary",),
        ),
    )(tau, scores, noise)


def _take16(x, idx):
    dn = lax.GatherDimensionNumbers(
        offset_dims=(), collapsed_slice_dims=(0,), start_index_map=(0,))
    return lax.gather(x, idx[:, None], dn, (1,),
                      mode=lax.GatherScatterMode.PROMISE_IN_BOUNDS)


def _sc_body(tau_hbm, scores_hbm, noise_hbm, out_hbm,
             key_buf, aux_buf, sent_buf, tau_buf):
    rows_per_w = _SC_ROWS // _NW
    wid = lax.axis_index("s") * 2 + lax.axis_index("c")
    iota = lax.iota(jnp.int32, 16)
    zero_i = jnp.zeros((16,), jnp.int32)
    one_i = jnp.full((16,), 1, jnp.int32)
    k_spl = jnp.full((16,), _K, jnp.int32)
    zero_f = jnp.zeros((16,), jnp.float32)
    one_f = jnp.full((16,), 1.0, jnp.float32)

    pltpu.sync_copy(tau_hbm, tau_buf)
    tauv = tau_buf[...]

    def allreduce(v):
        for d in (1, 2, 4, 8):
            v = v + _take16(v, iota ^ d)
        return v

    def key_at(v):
        return lax.bitcast_convert_type(key_buf[pl.ds(v * 16, 16)], jnp.int32)

    def count_ge(t_spl):
        def cbody(j, accs):
            accs = list(accs)
            for k in range(_UNROLL):
                kv = key_at(j * _UNROLL + k)
                accs[k] = accs[k] + jnp.where(kv >= t_spl, one_i, zero_i)
            return tuple(accs)
        accs = lax.fori_loop(0, _NVR // _UNROLL, cbody, (zero_i,) * _UNROLL)
        acc = accs[0]
        for k in range(1, _UNROLL):
            acc = acc + accs[k]
        return allreduce(acc)

    for i in range(rows_per_w):
        r = wid * rows_per_w + i
        pltpu.sync_copy(scores_hbm.at[r], key_buf)
        pltpu.sync_copy(noise_hbm.at[r], aux_buf)

        def p1(j, carry):
            for k in range(_UNROLL):
                sl = pl.ds((j * _UNROLL + k) * 16, 16)
                gv = key_buf[sl] / tauv + aux_buf[sl]
                u = lax.bitcast_convert_type(gv, jnp.int32)
                key = u ^ (lax.shift_right_arithmetic(u, 31) & 0x7FFFFFFF)
                key_buf[sl] = lax.bitcast_convert_type(key, jnp.float32)
            return carry
        lax.fori_loop(0, _NVR // _UNROLL, p1, 0)

        c0 = count_ge(zero_i)
        base = jnp.where(c0 >= k_spl, zero_i, jnp.full((16,), _INT_MIN,
                                                       jnp.int32))

        def vstep(it, b):
            cand = b + (jnp.int32(1) << (jnp.int32(30) - it))
            c = count_ge(cand)
            ok = jnp.logical_and(c >= k_spl, cand >= b)
            return jnp.where(ok, cand, b)
        thr = lax.fori_loop(0, 31, vstep, base)

        def p2(j, accs):
            accs = list(accs)
            for k in range(_UNROLL):
                v = j * _UNROLL + k
                kv = key_at(v)
                accs[k] = accs[k] + jnp.where(kv > thr, one_i, zero_i)
                idxv = iota + v * 16
                sent_buf[pl.ds(v * 16, 16)] = jnp.where(
                    kv == thr, idxv, jnp.full((16,), _BIG, jnp.int32))
            return tuple(accs)
        accs = lax.fori_loop(0, _NVR // _UNROLL, p2, (zero_i,) * _UNROLL)
        acc = accs[0]
        for k in range(1, _UNROLL):
            acc = acc + accs[k]
        need = k_spl - allreduce(acc)

        def istep(it, b):
            candI = b + (jnp.int32(1) << (jnp.int32(14) - it))

            def ibody(j, accs):
                accs = list(accs)
                for k in range(_UNROLL):
                    sv = sent_buf[pl.ds((j * _UNROLL + k) * 16, 16)]
                    accs[k] = accs[k] + jnp.where(sv <= candI, one_i, zero_i)
                return tuple(accs)
            accs = lax.fori_loop(0, _NVR // _UNROLL, ibody,
                                 (zero_i,) * _UNROLL)
            acc = accs[0]
            for k in range(1, _UNROLL):
                acc = acc + accs[k]
            c = allreduce(acc)
            return jnp.where(c <= need, candI, b)
        tieI = lax.fori_loop(0, 15, istep, jnp.full((16,), -1, jnp.int32))

        def p3(j, carry):
            for k in range(_UNROLL):
                v = j * _UNROLL + k
                kv = key_at(v)
                sv = sent_buf[pl.ds(v * 16, 16)]
                sel = jnp.logical_or(kv > thr, sv <= tieI)
                aux_buf[pl.ds(v * 16, 16)] = jnp.where(sel, one_f, zero_f)
            return carry
        lax.fori_loop(0, _NVR // _UNROLL, p3, 0)
        pltpu.sync_copy(aux_buf, out_hbm.at[r])


def _sc_call(scores, tau16, noise):
    mesh = plsc.VectorSubcoreMesh(core_axis_name="c", subcore_axis_name="s")
    f = pl.kernel(
        _sc_body,
        out_type=jax.ShapeDtypeStruct((_SC_ROWS, _COLS), jnp.float32),
        mesh=mesh,
        scratch_types=[
            pltpu.VMEM((_COLS,), jnp.float32),
            pltpu.VMEM((_COLS,), jnp.float32),
            pltpu.VMEM((_COLS,), jnp.int32),
            pltpu.VMEM((16,), jnp.float32),
        ],
    )
    return f(tau16, scores, noise)


def kernel(scores, tau):
    if _SC_ROWS == _ROWS:
        tau16 = jnp.broadcast_to(tau.astype(jnp.float32), (16,))
        return _sc_call(scores, tau16, _gumbel_noise(1))
    if _SC_ROWS == 0:
        return _tc_call(scores, tau, _gumbel_noise(0))
    tau16 = jnp.broadcast_to(tau.astype(jnp.float32), (16,))
    sc_out = _sc_call(scores[:_SC_ROWS], tau16, _gumbel_noise(1))
    tc_out = _tc_call(scores, tau, _gumbel_noise(0))
    return lax.dynamic_update_slice(tc_out, sc_out, (0, 0))
